# Initial kernel scaffold; baseline (speedup 1.0000x reference)
import jax
import jax.numpy as jnp
from jax.experimental import pallas as pl


def kernel(node_fea, edge_index, edge_attr, batch, atoms_batch, emb_W, emb_b, c0_W1, c0_b1, c0_W2, c0_b2, c0_W3, c0_b3, c0_bn_g, c0_bn_b, c1_W1, c1_b1, c1_W2, c1_b2, c1_W3, c1_b3, c1_bn_g, c1_bn_b, c2_W1, c2_b1, c2_W2, c2_b2, c2_W3, c2_b3, c2_bn_g, c2_bn_b, fc1_W, fc1_b, fc2_W, fc2_b, fc3_W, fc3_b):
    raise NotImplementedError("write your pallas kernel here")


# speedup vs baseline: 1.6458x; 1.6458x over previous
import functools

import jax
import jax.numpy as jnp
from jax import lax
from jax.experimental import pallas as pl
from jax.experimental.pallas import tpu as pltpu
from jax.experimental.pallas import tpu_sc as plsc

N = 10000
E = 160000
D = 128
DE = 16
NORB = 9
GRID = 256
NG = 64

NC = 2
NS = 16
NW = NC * NS

CH = 128
EC_PER_W = 40
E_PAD = NW * EC_PER_W * CH
TRASH = N
NACC = 10112
RPT = NACC // NS

A = N * NORB
AC = 704
AC_PER_T = 48
AC_PAD = NS * AC_PER_T
A_TAIL = A - (AC - 1) * CH
A_PAD = AC_PAD * CH

_mesh = plsc.VectorSubcoreMesh(core_axis_name="c", subcore_axis_name="s")
_f32 = jnp.float32


def _sp(x):
    return jnp.maximum(x, 0.0) + jnp.log(1.0 + jnp.exp(-jnp.abs(x)))


def _gather_edges(x, src_g, dst_g):
    @functools.partial(
        pl.kernel,
        out_type=jax.ShapeDtypeStruct((E_PAD, 2 * D), _f32),
        mesh=_mesh,
        scratch_types=[
            pltpu.VMEM((EC_PER_W, CH), jnp.int32),
            pltpu.VMEM((EC_PER_W, CH), jnp.int32),
            pltpu.VMEM((CH, D), _f32),
            pltpu.VMEM((CH, D), _f32),
            pltpu.SemaphoreType.DMA,
            pltpu.SemaphoreType.DMA,
        ],
    )
    def k(x_h, sg_h, dg_h, w2_h, idx_s, idx_d, bufs, bufd, sem1, sem2):
        w = lax.axis_index("s") * NC + lax.axis_index("c")
        pltpu.sync_copy(sg_h.at[pl.ds(w * EC_PER_W, EC_PER_W)], idx_s)
        pltpu.sync_copy(dg_h.at[pl.ds(w * EC_PER_W, EC_PER_W)], idx_d)

        def body(j, carry):
            cs = pltpu.async_copy(x_h.at[idx_s.at[j]], bufs, sem1)
            cd = pltpu.async_copy(x_h.at[idx_d.at[j]], bufd, sem2)
            cs.wait()
            cd.wait()
            row0 = (w * EC_PER_W + j) * CH
            pltpu.sync_copy(bufs, w2_h.at[pl.ds(row0, CH), pl.ds(0, D)])
            pltpu.sync_copy(bufd, w2_h.at[pl.ds(row0, CH), pl.ds(D, D)])
            return carry

        lax.fori_loop(0, EC_PER_W, body, 0)

    return k(x, src_g, dst_g)


def _scatter_edges(m, src_s, zeros):
    half_chunks = E_PAD // CH // NC

    @functools.partial(
        pl.kernel,
        out_type=jax.ShapeDtypeStruct((NC, NACC, D), _f32),
        mesh=_mesh,
        scratch_types=[
            pltpu.VMEM((EC_PER_W, CH), jnp.int32),
            pltpu.VMEM((CH, D), _f32),
            pltpu.VMEM_SHARED((NACC, D), _f32),
        ],
    )
    def k(m_h, ss_h, z_h, out_h, idxb, vbuf, acc):
        c = lax.axis_index("c")
        s = lax.axis_index("s")
        pltpu.sync_copy(z_h.at[pl.ds(s * RPT, RPT)], acc.at[pl.ds(s * RPT, RPT)])
        plsc.subcore_barrier()
        base = c * half_chunks + s * EC_PER_W
        pltpu.sync_copy(ss_h.at[pl.ds(base, EC_PER_W)], idxb)

        def body(j, carry):
            pltpu.sync_copy(m_h.at[pl.ds((base + j) * CH, CH)], vbuf)
            pltpu.sync_copy(vbuf, acc.at[idxb.at[j]], add=True)
            return carry

        lax.fori_loop(0, EC_PER_W, body, 0)
        plsc.subcore_barrier()
        pltpu.sync_copy(acc.at[pl.ds(s * RPT, RPT)],
                        out_h.at[c, pl.ds(s * RPT, RPT)])

    return k(m, src_s, zeros)


def _count_edges(src_s, zeros, ones128):
    half_chunks = E_PAD // CH // NC

    @functools.partial(
        pl.kernel,
        out_type=jax.ShapeDtypeStruct((NC, NACC, D), _f32),
        mesh=_mesh,
        scratch_types=[
            pltpu.VMEM((EC_PER_W, CH), jnp.int32),
            pltpu.VMEM((CH, D), _f32),
            pltpu.VMEM_SHARED((NACC, D), _f32),
        ],
    )
    def k(ss_h, z_h, o_h, out_h, idxb, ones_v, acc):
        c = lax.axis_index("c")
        s = lax.axis_index("s")
        pltpu.sync_copy(o_h, ones_v)
        pltpu.sync_copy(z_h.at[pl.ds(s * RPT, RPT)],
                        acc.at[pl.ds(s * RPT, RPT)])
        plsc.subcore_barrier()
        base = c * half_chunks + s * EC_PER_W
        pltpu.sync_copy(ss_h.at[pl.ds(base, EC_PER_W)], idxb)

        def body(j, carry):
            pltpu.sync_copy(ones_v, acc.at[idxb.at[j]], add=True)
            return carry

        lax.fori_loop(0, EC_PER_W, body, 0)
        plsc.subcore_barrier()
        pltpu.sync_copy(acc.at[pl.ds(s * RPT, RPT)],
                        out_h.at[c, pl.ds(s * RPT, RPT)])

    return k(src_s, zeros, ones128)


def _atomic_dos(pdos_r, ab_p, zeros):
    @functools.partial(
        pl.kernel,
        out_type=jax.ShapeDtypeStruct((NACC, GRID), _f32),
        mesh=_mesh,
        scratch_types=[
            pltpu.VMEM((AC_PER_T, CH), jnp.int32),
            pltpu.VMEM((CH, D), _f32),
            pltpu.VMEM_SHARED((NACC, D), _f32),
        ],
    )
    def k(p_h, ab_h, z_h, out_h, idxb, vbuf, acc):
        c = lax.axis_index("c")
        s = lax.axis_index("s")
        col0 = c * D
        pltpu.sync_copy(z_h.at[pl.ds(s * RPT, RPT)], acc.at[pl.ds(s * RPT, RPT)])
        plsc.subcore_barrier()
        pltpu.sync_copy(ab_h.at[pl.ds(s * AC_PER_T, AC_PER_T)], idxb)

        def body(j, carry):
            g = s * AC_PER_T + j

            @pl.when(g < AC - 1)
            def _full():
                pltpu.sync_copy(p_h.at[pl.ds(g * CH, CH), pl.ds(col0, D)], vbuf)

            @pl.when(g == AC - 1)
            def _tail():
                pltpu.sync_copy(p_h.at[pl.ds(g * CH, A_TAIL), pl.ds(col0, D)],
                                vbuf.at[pl.ds(0, A_TAIL)])

            pltpu.sync_copy(vbuf, acc.at[idxb.at[j]], add=True)
            return carry

        lax.fori_loop(0, AC_PER_T, body, 0)
        plsc.subcore_barrier()
        pltpu.sync_copy(acc.at[pl.ds(s * RPT, RPT)],
                        out_h.at[pl.ds(s * RPT, RPT), pl.ds(col0, D)])

    return k(pdos_r, ab_p, zeros)


def _embed(nf_pad, emb_W, emb_b):
    def body(nf_ref, w_ref, b_ref, o_ref):
        o_ref[...] = _sp(
            jnp.dot(nf_ref[...], w_ref[...], preferred_element_type=_f32)
            + b_ref[...])

    return pl.pallas_call(
        body, out_shape=jax.ShapeDtypeStruct((NACC, D), _f32),
    )(nf_pad, emb_W, emb_b.reshape(1, D))


def _edge_mlp(w2, ea_pad, W1ab, W1c, b1, W2, b2, W3, b3):
    BE = 512
    nblk = E_PAD // BE

    def body(w2_ref, ea_ref, w1ab_ref, w1c_ref, b1_ref, w2w_ref, b2_ref,
             w3_ref, b3_ref, m_ref):
        z = _sp(jnp.dot(w2_ref[...], w1ab_ref[...], preferred_element_type=_f32)
                + jnp.dot(ea_ref[...], w1c_ref[...], preferred_element_type=_f32)
                + b1_ref[...])
        z = _sp(jnp.dot(z, w2w_ref[...], preferred_element_type=_f32)
                + b2_ref[...])
        m_ref[...] = _sp(jnp.dot(z, w3_ref[...], preferred_element_type=_f32)
                         + b3_ref[...])

    full = lambda shape: pl.BlockSpec(shape, lambda i: (0, 0))
    return pl.pallas_call(
        body,
        grid=(nblk,),
        in_specs=[
            pl.BlockSpec((BE, 2 * D), lambda i: (i, 0)),
            pl.BlockSpec((BE, DE), lambda i: (i, 0)),
            full((2 * D, 256)),
            full((DE, 256)),
            full((1, 256)),
            full((256, 256)),
            full((1, 256)),
            full((256, D)),
            full((1, D)),
        ],
        out_specs=pl.BlockSpec((BE, D), lambda i: (i, 0)),
        out_shape=jax.ShapeDtypeStruct((E_PAD, D), _f32),
    )(w2, ea_pad, W1ab, W1c, b1.reshape(1, 256), W2, b2.reshape(1, 256),
      W3, b3.reshape(1, D))


def _bn_residual(parts, cnts, x, g, b):
    def body(p_ref, c_ref, x_ref, g_ref, b_ref, o_ref):
        s = p_ref[0] + p_ref[1]
        cnt = c_ref[0, :, 0:1] + c_ref[1, :, 0:1]
        agg = s / jnp.maximum(cnt, 1.0)
        rows = lax.broadcasted_iota(jnp.int32, (NACC, 1), 0)
        mask = rows < N
        aggm = jnp.where(mask, agg, 0.0)
        mu = jnp.sum(aggm, axis=0, keepdims=True) / N
        var = jnp.sum(jnp.where(mask, (agg - mu) ** 2, 0.0),
                      axis=0, keepdims=True) / N
        bn = g_ref[...] * (agg - mu) * lax.rsqrt(var + 1e-5) + b_ref[...]
        o_ref[...] = _sp(bn + x_ref[...])

    return pl.pallas_call(
        body, out_shape=jax.ShapeDtypeStruct((NACC, D), _f32),
    )(parts, cnts, x, g.reshape(1, D), b.reshape(1, D))


def _head(x, batch_r, fc1_W, fc1_b, fc2_W, fc2_b, fc3_W, fc3_b):
    BR = 400
    nblk = N // BR

    def body(x_ref, bt_ref, w1_ref, b1_ref, w2_ref, b2_ref, w3_ref, b3_ref,
             pdos_ref, dos_ref):
        h = _sp(jnp.dot(x_ref[...], w1_ref[...], preferred_element_type=_f32)
                + b1_ref[...])
        h = _sp(jnp.dot(h, w2_ref[...], preferred_element_type=_f32)
                + b2_ref[...])
        p = 1.0 / (1.0 + jnp.exp(
            -(jnp.dot(h, w3_ref[...], preferred_element_type=_f32)
              + b3_ref[...])))
        pdos_ref[...] = p
        pb = jnp.zeros((BR, GRID), _f32)
        for o in range(NORB):
            pb = pb + p[:, o * GRID:(o + 1) * GRID]
        bt = bt_ref[0]
        gids = lax.broadcasted_iota(jnp.int32, (NG, BR), 0)
        oh = (gids == bt).astype(_f32)
        contrib = jax.lax.dot_general(
            oh, pb, (((1,), (0,)), ((), ())), preferred_element_type=_f32,
            precision=lax.Precision.HIGHEST)

        @pl.when(pl.program_id(0) == 0)
        def _init():
            dos_ref[...] = jnp.zeros((NG, GRID), _f32)

        dos_ref[...] += contrib

    full = lambda shape: pl.BlockSpec(shape, lambda i: (0, 0))
    return pl.pallas_call(
        body,
        grid=(nblk,),
        in_specs=[
            pl.BlockSpec((BR, D), lambda i: (i, 0)),
            pl.BlockSpec((1, 1, BR), lambda i: (i, 0, 0)),
            full((D, 256)),
            full((1, 256)),
            full((256, 512)),
            full((1, 512)),
            full((512, NORB * GRID)),
            full((1, NORB * GRID)),
        ],
        out_specs=[
            pl.BlockSpec((BR, NORB * GRID), lambda i: (i, 0)),
            pl.BlockSpec((NG, GRID), lambda i: (0, 0)),
        ],
        out_shape=[
            jax.ShapeDtypeStruct((N, NORB * GRID), _f32),
            jax.ShapeDtypeStruct((NG, GRID), _f32),
        ],
    )(x, batch_r, fc1_W, fc1_b.reshape(1, 256), fc2_W, fc2_b.reshape(1, 512),
      fc3_W, fc3_b.reshape(1, NORB * GRID))


def kernel(node_fea, edge_index, edge_attr, batch, atoms_batch, emb_W, emb_b,
           c0_W1, c0_b1, c0_W2, c0_b2, c0_W3, c0_b3, c0_bn_g, c0_bn_b,
           c1_W1, c1_b1, c1_W2, c1_b2, c1_W3, c1_b3, c1_bn_g, c1_bn_b,
           c2_W1, c2_b1, c2_W2, c2_b2, c2_W3, c2_b3, c2_bn_g, c2_bn_b,
           fc1_W, fc1_b, fc2_W, fc2_b, fc3_W, fc3_b):
    src = edge_index[0]
    dst = edge_index[1]
    pad0 = jnp.zeros((E_PAD - E,), jnp.int32)
    src_g = jnp.concatenate([src, pad0]).reshape(E_PAD // CH, CH)
    dst_g = jnp.concatenate([dst, pad0]).reshape(E_PAD // CH, CH)
    padt = jnp.full((E_PAD - E,), TRASH, jnp.int32)
    src_s = jnp.concatenate([src, padt]).reshape(E_PAD // CH, CH)
    ea_pad = jnp.concatenate(
        [edge_attr, jnp.zeros((E_PAD - E, DE), _f32)], axis=0)
    ab_p = jnp.concatenate(
        [atoms_batch, jnp.full((A_PAD - A,), TRASH, jnp.int32)]
    ).reshape(AC_PAD, CH)
    nf_pad = jnp.concatenate([node_fea, jnp.zeros((NACC - N, D), _f32)], axis=0)
    zeros = jnp.zeros((NACC, D), _f32)
    ones128 = jnp.ones((CH, D), _f32)
    batch_r = batch.reshape(N // 400, 1, 400)

    layers = [
        (c0_W1, c0_b1, c0_W2, c0_b2, c0_W3, c0_b3, c0_bn_g, c0_bn_b),
        (c1_W1, c1_b1, c1_W2, c1_b2, c1_W3, c1_b3, c1_bn_g, c1_bn_b),
        (c2_W1, c2_b1, c2_W2, c2_b2, c2_W3, c2_b3, c2_bn_g, c2_bn_b),
    ]

    x = _embed(nf_pad, emb_W, emb_b)
    cnts = _count_edges(src_s, zeros, ones128)
    for (W1, b1, W2, b2, W3, b3, bn_g, bn_b) in layers:
        w2 = _gather_edges(x, src_g, dst_g)
        m = _edge_mlp(w2, ea_pad, W1[:2 * D], W1[2 * D:], b1, W2, b2, W3, b3)
        parts = _scatter_edges(m, src_s, zeros)
        x = _bn_residual(parts, cnts, x, bn_g, bn_b)

    pdos, dos = _head(x, batch_r, fc1_W, fc1_b, fc2_W, fc2_b, fc3_W, fc3_b)
    pdos_r = pdos.reshape(A, GRID)
    ados = _atomic_dos(pdos_r, ab_p, zeros)[:N]
    return pdos_r, ados, dos


# speedup vs baseline: 1.8591x; 1.1296x over previous
import functools

import jax
import jax.numpy as jnp
from jax import lax
from jax.experimental import pallas as pl
from jax.experimental.pallas import tpu as pltpu
from jax.experimental.pallas import tpu_sc as plsc

N = 10000
E = 160000
D = 128
DE = 16
NORB = 9
GRID = 256
NG = 64

NC = 2
NS = 16
NW = NC * NS

CH = 128
EC_PER_W = 40
E_PAD = NW * EC_PER_W * CH
TRASH = N
NACC = 10112
RPT = NACC // NS

A = N * NORB
AC = 704
AC_PER_T = 48
AC_PAD = NS * AC_PER_T
A_TAIL = A - (AC - 1) * CH
A_PAD = AC_PAD * CH

_mesh = plsc.VectorSubcoreMesh(core_axis_name="c", subcore_axis_name="s")
_f32 = jnp.float32


def _stage(ecw):
    return ecw if ecw % 8 == 0 else 2 * ecw


def _sp(x):
    return jnp.maximum(x, 0.0) + jnp.log(1.0 + jnp.exp(-jnp.abs(x)))


GCH = CH


def _gather_edges(x, src_g, dst_g, ecw):
    NB = 2
    n_e = NW * ecw * GCH

    @functools.partial(
        pl.kernel,
        out_type=jax.ShapeDtypeStruct((n_e, 2 * D), _f32),
        mesh=_mesh,
        scratch_types=(
            [pltpu.VMEM((_stage(ecw), GCH), jnp.int32)] * 2
            + [pltpu.VMEM((GCH, D), _f32)] * (2 * NB)
            + [pltpu.SemaphoreType.DMA] * (2 * NB)
        ),
    )
    def k(x_h, sg_h, dg_h, w2_h, idx_s, idx_d, *bufsem):
        bufs = bufsem[:2 * NB]
        sems = bufsem[2 * NB:]
        w = lax.axis_index("s") * NC + lax.axis_index("c")
        if ecw % 8 == 0:
            joff = 0
            pltpu.sync_copy(sg_h.at[pl.ds(w * ecw, ecw)], idx_s)
            pltpu.sync_copy(dg_h.at[pl.ds(w * ecw, ecw)], idx_d)
        else:
            joff = (w % 2) * ecw
            pltpu.sync_copy(
                sg_h.at[pl.ds((w // 2) * 2 * ecw, 2 * ecw)], idx_s)
            pltpu.sync_copy(
                dg_h.at[pl.ds((w // 2) * 2 * ecw, 2 * ecw)], idx_d)

        def start_g(j, b):
            pltpu.async_copy(x_h.at[idx_s.at[j + joff]], bufs[2 * b],
                             sems[2 * b])
            pltpu.async_copy(x_h.at[idx_d.at[j + joff]], bufs[2 * b + 1],
                             sems[2 * b + 1])

        def wait_g(j, b):
            pltpu.make_async_copy(
                x_h.at[idx_s.at[j + joff]], bufs[2 * b], sems[2 * b]).wait()
            pltpu.make_async_copy(
                x_h.at[idx_d.at[j + joff]], bufs[2 * b + 1],
                sems[2 * b + 1]).wait()

        def start_w(j, b):
            row0 = (w * ecw + j) * GCH
            pltpu.async_copy(bufs[2 * b],
                             w2_h.at[pl.ds(row0, GCH), pl.ds(0, D)],
                             sems[2 * b])
            pltpu.async_copy(bufs[2 * b + 1],
                             w2_h.at[pl.ds(row0, GCH), pl.ds(D, D)],
                             sems[2 * b + 1])

        def wait_w(j, b):
            row0 = (w * ecw + j) * GCH
            pltpu.make_async_copy(
                bufs[2 * b], w2_h.at[pl.ds(row0, GCH), pl.ds(0, D)],
                sems[2 * b]).wait()
            pltpu.make_async_copy(
                bufs[2 * b + 1], w2_h.at[pl.ds(row0, GCH), pl.ds(D, D)],
                sems[2 * b + 1]).wait()

        for b in range(NB):
            start_g(b, b)

        def body(h, carry):
            j0 = NB * h
            for b in range(NB):
                wait_g(j0 + b, b)
                start_w(j0 + b, b)
            for b in range(NB):
                wait_w(j0 + b, b)

                @pl.when(h < ecw // NB - 1)
                def _next():
                    start_g(j0 + NB + b, b)

            return carry

        lax.fori_loop(0, ecw // NB, body, 0)

    return k(x, src_g, dst_g)


def _scatter_edges(m, src_s, zeros, ecw):
    half_chunks = NW * ecw // NC

    @functools.partial(
        pl.kernel,
        out_type=jax.ShapeDtypeStruct((NC, NACC, D), _f32),
        mesh=_mesh,
        scratch_types=[
            pltpu.VMEM((_stage(ecw), CH), jnp.int32),
            pltpu.VMEM((CH, D), _f32),
            pltpu.VMEM((CH, D), _f32),
            pltpu.VMEM_SHARED((NACC, D), _f32),
        ],
    )
    def k(m_h, ss_h, z_h, out_h, idxb, v0, v1, acc):
        c = lax.axis_index("c")
        s = lax.axis_index("s")
        pltpu.sync_copy(z_h.at[pl.ds(s * RPT, RPT)], acc.at[pl.ds(s * RPT, RPT)])
        plsc.subcore_barrier()
        base = c * half_chunks + s * ecw
        if ecw % 8 == 0:
            joff = 0
            pltpu.sync_copy(ss_h.at[pl.ds(base, ecw)], idxb)
        else:
            joff = (s % 2) * ecw
            pltpu.sync_copy(
                ss_h.at[pl.ds(c * half_chunks + (s // 2) * 2 * ecw, 2 * ecw)],
                idxb)

        def run_pipe(l0, l1, s0, s1):
            def start_l(j, vb, sem):
                pltpu.async_copy(m_h.at[pl.ds((base + j) * CH, CH)], vb, sem)

            def wait_l(j, vb, sem):
                pltpu.make_async_copy(
                    m_h.at[pl.ds((base + j) * CH, CH)], vb, sem).wait()

            def start_s(j, vb, sem):
                pltpu.async_copy(vb, acc.at[idxb.at[j + joff]], sem, add=True)

            def wait_s(j, vb, sem):
                pltpu.make_async_copy(
                    vb, acc.at[idxb.at[j + joff]], sem).wait()

            start_l(0, v0, l0)

            def body(h, carry):
                j0 = 2 * h
                j1 = 2 * h + 1
                start_l(j1, v1, l1)
                wait_l(j0, v0, l0)
                start_s(j0, v0, s0)
                wait_l(j1, v1, l1)
                start_s(j1, v1, s1)
                wait_s(j0, v0, s0)

                @pl.when(h < ecw // 2 - 1)
                def _next():
                    start_l(j0 + 2, v0, l0)

                wait_s(j1, v1, s1)
                return carry

            lax.fori_loop(0, ecw // 2, body, 0)

        pl.run_scoped(
            run_pipe,
            pltpu.SemaphoreType.DMA, pltpu.SemaphoreType.DMA,
            pltpu.SemaphoreType.DMA, pltpu.SemaphoreType.DMA)
        plsc.subcore_barrier()
        pltpu.sync_copy(acc.at[pl.ds(s * RPT, RPT)],
                        out_h.at[c, pl.ds(s * RPT, RPT)])

    return k(m, src_s, zeros)


def _count_edges(src_s, zeros, ones128):
    half_chunks = E_PAD // CH // NC

    @functools.partial(
        pl.kernel,
        out_type=jax.ShapeDtypeStruct((NC, NACC, D), _f32),
        mesh=_mesh,
        scratch_types=[
            pltpu.VMEM((EC_PER_W, CH), jnp.int32),
            pltpu.VMEM((CH, D), _f32),
            pltpu.VMEM_SHARED((NACC, D), _f32),
        ],
    )
    def k(ss_h, z_h, o_h, out_h, idxb, ones_v, acc):
        c = lax.axis_index("c")
        s = lax.axis_index("s")
        pltpu.sync_copy(o_h, ones_v)
        pltpu.sync_copy(z_h.at[pl.ds(s * RPT, RPT)],
                        acc.at[pl.ds(s * RPT, RPT)])
        plsc.subcore_barrier()
        base = c * half_chunks + s * EC_PER_W
        pltpu.sync_copy(ss_h.at[pl.ds(base, EC_PER_W)], idxb)

        def run_sc(sem):
            def fire(j, carry):
                pltpu.async_copy(ones_v, acc.at[idxb.at[j]], sem, add=True)
                return carry

            lax.fori_loop(0, EC_PER_W, fire, 0)

            def drain(j, carry):
                pltpu.make_async_copy(ones_v, acc.at[idxb.at[j]], sem).wait()
                return carry

            lax.fori_loop(0, EC_PER_W, drain, 0)

        pl.run_scoped(run_sc, pltpu.SemaphoreType.DMA)
        plsc.subcore_barrier()
        pltpu.sync_copy(acc.at[pl.ds(s * RPT, RPT)],
                        out_h.at[c, pl.ds(s * RPT, RPT)])

    return k(src_s, zeros, ones128)


def _atomic_dos(pdos_r, ab_p, zeros):
    @functools.partial(
        pl.kernel,
        out_type=jax.ShapeDtypeStruct((NACC, GRID), _f32),
        mesh=_mesh,
        scratch_types=[
            pltpu.VMEM((AC_PER_T, CH), jnp.int32),
            pltpu.VMEM((CH, D), _f32),
            pltpu.VMEM((CH, D), _f32),
            pltpu.VMEM_SHARED((NACC, D), _f32),
        ],
    )
    def k(p_h, ab_h, z_h, out_h, idxb, v0, v1, acc):
        c = lax.axis_index("c")
        s = lax.axis_index("s")
        col0 = c * D
        pltpu.sync_copy(z_h.at[pl.ds(s * RPT, RPT)], acc.at[pl.ds(s * RPT, RPT)])
        plsc.subcore_barrier()
        pltpu.sync_copy(ab_h.at[pl.ds(s * AC_PER_T, AC_PER_T)], idxb)

        def row0_of(j):
            g = s * AC_PER_T + j
            return jnp.where(g <= AC - 2, g * CH, A - CH)

        def run_pipe(l0, l1, s0, s1):
            def start_l(j, vb, sem):
                pltpu.async_copy(
                    p_h.at[pl.ds(row0_of(j), CH), pl.ds(col0, D)], vb, sem)

            def wait_l(j, vb, sem):
                pltpu.make_async_copy(
                    p_h.at[pl.ds(row0_of(j), CH), pl.ds(col0, D)], vb,
                    sem).wait()

            def start_s(j, vb, sem):
                pltpu.async_copy(vb, acc.at[idxb.at[j]], sem, add=True)

            def wait_s(j, vb, sem):
                pltpu.make_async_copy(vb, acc.at[idxb.at[j]], sem).wait()

            start_l(0, v0, l0)

            def body(h, carry):
                j0 = 2 * h
                j1 = 2 * h + 1
                start_l(j1, v1, l1)
                wait_l(j0, v0, l0)
                start_s(j0, v0, s0)
                wait_l(j1, v1, l1)
                start_s(j1, v1, s1)
                wait_s(j0, v0, s0)

                @pl.when(h < AC_PER_T // 2 - 1)
                def _next():
                    start_l(j0 + 2, v0, l0)

                wait_s(j1, v1, s1)
                return carry

            lax.fori_loop(0, AC_PER_T // 2, body, 0)

        pl.run_scoped(
            run_pipe,
            pltpu.SemaphoreType.DMA, pltpu.SemaphoreType.DMA,
            pltpu.SemaphoreType.DMA, pltpu.SemaphoreType.DMA)
        plsc.subcore_barrier()
        pltpu.sync_copy(acc.at[pl.ds(s * RPT, RPT)],
                        out_h.at[pl.ds(s * RPT, RPT), pl.ds(col0, D)])

    return k(pdos_r, ab_p, zeros)


def _embed(nf_pad, emb_W, emb_b):
    def body(nf_ref, w_ref, b_ref, o_ref):
        o_ref[...] = _sp(
            jnp.dot(nf_ref[...], w_ref[...], preferred_element_type=_f32)
            + b_ref[...])

    return pl.pallas_call(
        body, out_shape=jax.ShapeDtypeStruct((NACC, D), _f32),
    )(nf_pad, emb_W, emb_b.reshape(1, D))


def _edge_mlp(w2, ea_pad, W1ab, W1c, b1, W2, b2, W3, b3):
    BE = 512
    n_e = w2.shape[0]
    nblk = n_e // BE

    def body(w2_ref, ea_ref, w1ab_ref, w1c_ref, b1_ref, w2w_ref, b2_ref,
             w3_ref, b3_ref, m_ref):
        z = _sp(jnp.dot(w2_ref[...], w1ab_ref[...], preferred_element_type=_f32)
                + jnp.dot(ea_ref[...], w1c_ref[...], preferred_element_type=_f32)
                + b1_ref[...])
        z = _sp(jnp.dot(z, w2w_ref[...], preferred_element_type=_f32)
                + b2_ref[...])
        m_ref[...] = _sp(jnp.dot(z, w3_ref[...], preferred_element_type=_f32)
                         + b3_ref[...])

    full = lambda shape: pl.BlockSpec(shape, lambda i: (0, 0))
    return pl.pallas_call(
        body,
        grid=(nblk,),
        in_specs=[
            pl.BlockSpec((BE, 2 * D), lambda i: (i, 0)),
            pl.BlockSpec((BE, DE), lambda i: (i, 0)),
            full((2 * D, 256)),
            full((DE, 256)),
            full((1, 256)),
            full((256, 256)),
            full((1, 256)),
            full((256, D)),
            full((1, D)),
        ],
        out_specs=pl.BlockSpec((BE, D), lambda i: (i, 0)),
        out_shape=jax.ShapeDtypeStruct((n_e, D), _f32),
    )(w2, ea_pad, W1ab, W1c, b1.reshape(1, 256), W2, b2.reshape(1, 256),
      W3, b3.reshape(1, D))


def _bn_residual(parts_a, parts_b, cnts, x, g, b):
    def body(p_ref, q_ref, c_ref, x_ref, g_ref, b_ref, o_ref):
        s = p_ref[0] + p_ref[1] + q_ref[0] + q_ref[1]
        cnt = c_ref[0, :, 0:1] + c_ref[1, :, 0:1]
        agg = s / jnp.maximum(cnt, 1.0)
        rows = lax.broadcasted_iota(jnp.int32, (NACC, 1), 0)
        mask = rows < N
        aggm = jnp.where(mask, agg, 0.0)
        mu = jnp.sum(aggm, axis=0, keepdims=True) / N
        var = jnp.sum(jnp.where(mask, (agg - mu) ** 2, 0.0),
                      axis=0, keepdims=True) / N
        bn = g_ref[...] * (agg - mu) * lax.rsqrt(var + 1e-5) + b_ref[...]
        o_ref[...] = _sp(bn + x_ref[...])

    return pl.pallas_call(
        body, out_shape=jax.ShapeDtypeStruct((NACC, D), _f32),
    )(parts_a, parts_b, cnts, x, g.reshape(1, D), b.reshape(1, D))


def _head(x, batch_r, fc1_W, fc1_b, fc2_W, fc2_b, fc3_W, fc3_b):
    BR = 400
    nblk = N // BR

    def body(x_ref, bt_ref, w1_ref, b1_ref, w2_ref, b2_ref, w3_ref, b3_ref,
             pdos_ref, dos_ref):
        h = _sp(jnp.dot(x_ref[...], w1_ref[...], preferred_element_type=_f32)
                + b1_ref[...])
        h = _sp(jnp.dot(h, w2_ref[...], preferred_element_type=_f32)
                + b2_ref[...])
        p = 1.0 / (1.0 + jnp.exp(
            -(jnp.dot(h, w3_ref[...], preferred_element_type=_f32)
              + b3_ref[...])))
        pdos_ref[...] = p
        pb = jnp.zeros((BR, GRID), _f32)
        for o in range(NORB):
            pb = pb + p[:, o * GRID:(o + 1) * GRID]
        bt = bt_ref[0]
        gids = lax.broadcasted_iota(jnp.int32, (NG, BR), 0)
        oh = (gids == bt).astype(_f32)
        contrib = jax.lax.dot_general(
            oh, pb, (((1,), (0,)), ((), ())), preferred_element_type=_f32,
            precision=lax.Precision.HIGHEST)

        @pl.when(pl.program_id(0) == 0)
        def _init():
            dos_ref[...] = jnp.zeros((NG, GRID), _f32)

        dos_ref[...] += contrib

    full = lambda shape: pl.BlockSpec(shape, lambda i: (0, 0))
    return pl.pallas_call(
        body,
        grid=(nblk,),
        in_specs=[
            pl.BlockSpec((BR, D), lambda i: (i, 0)),
            pl.BlockSpec((1, 1, BR), lambda i: (i, 0, 0)),
            full((D, 256)),
            full((1, 256)),
            full((256, 512)),
            full((1, 512)),
            full((512, NORB * GRID)),
            full((1, NORB * GRID)),
        ],
        out_specs=[
            pl.BlockSpec((BR, NORB * GRID), lambda i: (i, 0)),
            pl.BlockSpec((NG, GRID), lambda i: (0, 0)),
        ],
        out_shape=[
            jax.ShapeDtypeStruct((N, NORB * GRID), _f32),
            jax.ShapeDtypeStruct((NG, GRID), _f32),
        ],
    )(x, batch_r, fc1_W, fc1_b.reshape(1, 256), fc2_W, fc2_b.reshape(1, 512),
      fc3_W, fc3_b.reshape(1, NORB * GRID))


def kernel(node_fea, edge_index, edge_attr, batch, atoms_batch, emb_W, emb_b,
           c0_W1, c0_b1, c0_W2, c0_b2, c0_W3, c0_b3, c0_bn_g, c0_bn_b,
           c1_W1, c1_b1, c1_W2, c1_b2, c1_W3, c1_b3, c1_bn_g, c1_bn_b,
           c2_W1, c2_b1, c2_W2, c2_b2, c2_W3, c2_b3, c2_bn_g, c2_bn_b,
           fc1_W, fc1_b, fc2_W, fc2_b, fc3_W, fc3_b):
    src = edge_index[0]
    dst = edge_index[1]
    pad0 = jnp.zeros((E_PAD - E,), jnp.int32)
    src_g = jnp.concatenate([src, pad0]).reshape(E_PAD // CH, CH)
    dst_g = jnp.concatenate([dst, pad0]).reshape(E_PAD // CH, CH)
    padt = jnp.full((E_PAD - E,), TRASH, jnp.int32)
    src_s = jnp.concatenate([src, padt]).reshape(E_PAD // CH, CH)
    ea_pad = jnp.concatenate(
        [edge_attr, jnp.zeros((E_PAD - E, DE), _f32)], axis=0)
    ab_p = jnp.concatenate([
        atoms_batch[:(AC - 1) * CH],
        jnp.full((CH - A_TAIL,), TRASH, jnp.int32),
        atoms_batch[(AC - 1) * CH:],
        jnp.full(((AC_PAD - AC) * CH,), TRASH, jnp.int32),
    ]).reshape(AC_PAD, CH)
    nf_pad = jnp.concatenate([node_fea, jnp.zeros((NACC - N, D), _f32)], axis=0)
    zeros = jnp.zeros((NACC, D), _f32)
    ones128 = jnp.ones((CH, D), _f32)
    batch_r = batch.reshape(N // 400, 1, 400)

    layers = [
        (c0_W1, c0_b1, c0_W2, c0_b2, c0_W3, c0_b3, c0_bn_g, c0_bn_b),
        (c1_W1, c1_b1, c1_W2, c1_b2, c1_W3, c1_b3, c1_bn_g, c1_bn_b),
        (c2_W1, c2_b1, c2_W2, c2_b2, c2_W3, c2_b3, c2_bn_g, c2_bn_b),
    ]

    HC = E_PAD // CH // 2
    ECW_H = HC // NW
    E_H = E_PAD // 2
    src_ga, src_gb = src_g[:HC], src_g[HC:]
    dst_ga, dst_gb = dst_g[:HC], dst_g[HC:]
    src_sa, src_sb = src_s[:HC], src_s[HC:]
    ea_a, ea_b = ea_pad[:E_H], ea_pad[E_H:]

    x = _embed(nf_pad, emb_W, emb_b)
    cnts = _count_edges(src_s, zeros, ones128)
    for (W1, b1, W2, b2, W3, b3, bn_g, bn_b) in layers:
        W1ab, W1c = W1[:2 * D], W1[2 * D:]
        w2a = _gather_edges(x, src_ga, dst_ga, ECW_H)
        w2b = _gather_edges(x, src_gb, dst_gb, ECW_H)
        ma = _edge_mlp(w2a, ea_a, W1ab, W1c, b1, W2, b2, W3, b3)
        mb = _edge_mlp(w2b, ea_b, W1ab, W1c, b1, W2, b2, W3, b3)
        pa = _scatter_edges(ma, src_sa, zeros, ECW_H)
        pb = _scatter_edges(mb, src_sb, zeros, ECW_H)
        x = _bn_residual(pa, pb, cnts, x, bn_g, bn_b)

    pdos, dos = _head(x, batch_r, fc1_W, fc1_b, fc2_W, fc2_b, fc3_W, fc3_b)
    pdos_r = pdos.reshape(A, GRID)
    ados = _atomic_dos(pdos_r, ab_p, zeros)[:N]
    return pdos_r, ados, dos
